# Initial kernel scaffold; baseline (speedup 1.0000x reference)
import jax
import jax.numpy as jnp
from jax.experimental import pallas as pl


def kernel(x, weight, dictionary_vector_indices, updated_weights):
    raise NotImplementedError("write your pallas kernel here")


# speedup vs baseline: 1.4141x; 1.4141x over previous
import functools

import jax
import jax.numpy as jnp
from jax import lax
from jax.experimental import pallas as pl


def _main_body(w_ref, x_ref, keep_ref, o_ref):
    i = pl.program_id(0)
    w = w_ref[...]
    n2 = jnp.sum(w * w, axis=0, keepdims=True)
    s = keep_ref[...] / jnp.maximum(jnp.sqrt(n2), 1e-8)
    xs = x_ref[...] * s
    part = lax.dot_general(xs, w, (((1,), (1,)), ((), ())),
                           preferred_element_type=jnp.float32,
                           precision=lax.Precision.HIGHEST)

    @pl.when(i == 0)
    def _init():
        o_ref[...] = part

    @pl.when(i > 0)
    def _acc():
        o_ref[...] += part


def _corr_body(u_ref, xg_ref, acc_ref, o_ref):
    i = pl.program_id(0)
    u = u_ref[...]
    n2 = jnp.sum(u * u, axis=0, keepdims=True)
    s = 1.0 / jnp.maximum(jnp.sqrt(n2), 1e-8)
    xs = xg_ref[...] * s
    part = lax.dot_general(xs, u, (((1,), (1,)), ((), ())),
                           preferred_element_type=jnp.float32,
                           precision=lax.Precision.HIGHEST)

    @pl.when(i == 0)
    def _init():
        o_ref[...] = acc_ref[...] + part

    @pl.when(i > 0)
    def _acc():
        o_ref[...] += part


def kernel(x, weight, dictionary_vector_indices, updated_weights):
    B, L = x.shape
    D = weight.shape[0]
    ND = updated_weights.shape[1]
    idx = dictionary_vector_indices.astype(jnp.int32)

    j = jnp.arange(ND, dtype=jnp.int32)
    arr = jnp.full((L,), -1, dtype=jnp.int32).at[idx].max(j)
    win = (arr[idx] == j).astype(jnp.float32)
    keep = (arr == -1).astype(jnp.float32)
    xg = x[:, idx] * win[None, :]

    LB = 512
    nL = L // LB
    keep3 = keep.reshape(nL, 1, LB)
    acc1 = pl.pallas_call(
        _main_body,
        grid=(nL,),
        in_specs=[
            pl.BlockSpec((D, LB), lambda i: (0, i)),
            pl.BlockSpec((B, LB), lambda i: (0, i)),
            pl.BlockSpec((None, 1, LB), lambda i: (i, 0, 0)),
        ],
        out_specs=pl.BlockSpec((B, D), lambda i: (0, 0)),
        out_shape=jax.ShapeDtypeStruct((B, D), jnp.float32),
    )(weight, x, keep3)

    NB = 512
    nN = ND // NB
    out = pl.pallas_call(
        _corr_body,
        grid=(nN,),
        in_specs=[
            pl.BlockSpec((D, NB), lambda i: (0, i)),
            pl.BlockSpec((B, NB), lambda i: (0, i)),
            pl.BlockSpec((B, D), lambda i: (0, 0)),
        ],
        out_specs=pl.BlockSpec((B, D), lambda i: (0, 0)),
        out_shape=jax.ShapeDtypeStruct((B, D), jnp.float32),
    )(updated_weights, xg, acc1)
    return out


# speedup vs baseline: 2.9288x; 2.0711x over previous
import functools

import jax
import jax.numpy as jnp
from jax import lax
from jax.experimental import pallas as pl


def _main_body(w_ref, x_ref, keep_ref, o_ref):
    i = pl.program_id(0)
    w = w_ref[...]
    n2 = jnp.sum(w * w, axis=0, keepdims=True)
    s = keep_ref[...] / jnp.maximum(jnp.sqrt(n2), 1e-8)
    xs = x_ref[...] * s
    part = lax.dot_general(xs, w, (((1,), (1,)), ((), ())),
                           preferred_element_type=jnp.float32,
                           precision=lax.Precision.DEFAULT)

    @pl.when(i == 0)
    def _init():
        o_ref[...] = part

    @pl.when(i > 0)
    def _acc():
        o_ref[...] += part


def _corr_body(u_ref, xg_ref, acc_ref, o_ref):
    i = pl.program_id(0)
    u = u_ref[...]
    n2 = jnp.sum(u * u, axis=0, keepdims=True)
    s = 1.0 / jnp.maximum(jnp.sqrt(n2), 1e-8)
    xs = xg_ref[...] * s
    part = lax.dot_general(xs, u, (((1,), (1,)), ((), ())),
                           preferred_element_type=jnp.float32,
                           precision=lax.Precision.DEFAULT)

    @pl.when(i == 0)
    def _init():
        o_ref[...] = acc_ref[...] + part

    @pl.when(i > 0)
    def _acc():
        o_ref[...] += part


def kernel(x, weight, dictionary_vector_indices, updated_weights):
    B, L = x.shape
    D = weight.shape[0]
    ND = updated_weights.shape[1]
    idx = dictionary_vector_indices.astype(jnp.int32)

    j = jnp.arange(ND, dtype=jnp.int32)
    arr = jnp.full((L,), -1, dtype=jnp.int32).at[idx].max(j)
    win = (arr[idx] == j).astype(jnp.float32)
    keep = (arr == -1).astype(jnp.float32)
    xg = x[:, idx] * win[None, :]

    LB = 512
    nL = L // LB
    keep3 = keep.reshape(nL, 1, LB)
    acc1 = pl.pallas_call(
        _main_body,
        grid=(nL,),
        in_specs=[
            pl.BlockSpec((D, LB), lambda i: (0, i)),
            pl.BlockSpec((B, LB), lambda i: (0, i)),
            pl.BlockSpec((None, 1, LB), lambda i: (i, 0, 0)),
        ],
        out_specs=pl.BlockSpec((B, D), lambda i: (0, 0)),
        out_shape=jax.ShapeDtypeStruct((B, D), jnp.float32),
    )(weight, x, keep3)

    NB = 512
    nN = ND // NB
    out = pl.pallas_call(
        _corr_body,
        grid=(nN,),
        in_specs=[
            pl.BlockSpec((D, NB), lambda i: (0, i)),
            pl.BlockSpec((B, NB), lambda i: (0, i)),
            pl.BlockSpec((B, D), lambda i: (0, 0)),
        ],
        out_specs=pl.BlockSpec((B, D), lambda i: (0, 0)),
        out_shape=jax.ShapeDtypeStruct((B, D), jnp.float32),
    )(updated_weights, xg, acc1)
    return out


# speedup vs baseline: 3.4511x; 1.1783x over previous
import functools

import jax
import jax.numpy as jnp
from jax import lax
from jax.experimental import pallas as pl
from jax.experimental.pallas import tpu as pltpu
from jax.experimental.pallas import tpu_sc as plsc

B = 128
L = 32768
D = 2048
ND = 4096

_NC = 2
_NS = 16
_NW = _NC * _NS
_SH = L // _NS
_ROWS = B // _NW
_NCH = ND // 16


def _route_body(idx_hbm, x_hbm, keep_hbm, xg_hbm,
                idx_v, shard_v, arr_v, win_v, keep_v, xrow_v, xgrow_v,
                shared_arr):
    c = lax.axis_index("c")
    s = lax.axis_index("s")
    wid = s * _NC + c
    base = s * _SH

    pltpu.sync_copy(idx_hbm, idx_v)

    def init_body(i, carry):
        shard_v[pl.ds(i * 16, 16)] = jnp.full((16,), -1, jnp.int32)
        return carry
    lax.fori_loop(0, _SH // 16, init_body, 0)

    def chunk_body(ci, carry):
        k16 = idx_v[pl.ds(ci * 16, 16)]
        j16 = lax.iota(jnp.int32, 16) + ci * 16
        m = (k16 >= base) & (k16 < base + _SH)
        loc = jnp.clip(k16 - base, 0, _SH - 1)
        cand = jnp.where(m, j16, -1)
        for _ in range(3):
            g = plsc.load_gather(shard_v, [loc], mask=m)
            need = m & (cand > g)
            plsc.store_scatter(shard_v, [loc], cand, mask=need)
        return carry
    lax.fori_loop(0, _NCH, chunk_body, 0)

    pltpu.sync_copy(shard_v, shared_arr.at[pl.ds(base, _SH)])
    plsc.subcore_barrier()
    pltpu.sync_copy(shared_arr, arr_v)

    kbase = base + c * (_SH // 2)
    def keep_body(i, carry):
        a16 = arr_v[pl.ds(kbase + i * 16, 16)]
        keep_v[pl.ds(i * 16, 16)] = jnp.where(
            a16 == -1, jnp.float32(1.0), jnp.float32(0.0))
        return carry
    lax.fori_loop(0, _SH // 32, keep_body, 0)
    pltpu.sync_copy(keep_v, keep_hbm.at[pl.ds(kbase, _SH // 2)])

    def win_body(ci, carry):
        k16 = idx_v[pl.ds(ci * 16, 16)]
        a = plsc.load_gather(arr_v, [k16])
        j16 = lax.iota(jnp.int32, 16) + ci * 16
        win_v[pl.ds(ci * 16, 16)] = jnp.where(
            a == j16, jnp.float32(1.0), jnp.float32(0.0))
        return carry
    lax.fori_loop(0, _NCH, win_body, 0)

    for r in range(_ROWS):
        b = wid * _ROWS + r
        pltpu.sync_copy(x_hbm.at[b], xrow_v)

        def gather_body(ci, carry):
            k16 = idx_v[pl.ds(ci * 16, 16)]
            g = plsc.load_gather(xrow_v, [k16])
            xgrow_v[pl.ds(ci * 16, 16)] = g * win_v[pl.ds(ci * 16, 16)]
            return carry
        lax.fori_loop(0, _NCH, gather_body, 0)
        pltpu.sync_copy(xgrow_v, xg_hbm.at[b])


_route = pl.kernel(
    _route_body,
    out_type=(
        jax.ShapeDtypeStruct((L,), jnp.float32),
        jax.ShapeDtypeStruct((B, ND), jnp.float32),
    ),
    mesh=plsc.VectorSubcoreMesh(core_axis_name="c", subcore_axis_name="s"),
    compiler_params=pltpu.CompilerParams(needs_layout_passes=False),
    scratch_types=(
        pltpu.VMEM((ND,), jnp.int32),
        pltpu.VMEM((_SH,), jnp.int32),
        pltpu.VMEM((L,), jnp.int32),
        pltpu.VMEM((ND,), jnp.float32),
        pltpu.VMEM((_SH // 2,), jnp.float32),
        pltpu.VMEM((L,), jnp.float32),
        pltpu.VMEM((ND,), jnp.float32),
        pltpu.VMEM_SHARED((L,), jnp.int32),
    ),
)


def _main_body(w_ref, x_ref, keep_ref, o_ref):
    i = pl.program_id(0)
    w = w_ref[...]
    n2 = jnp.sum(w * w, axis=0, keepdims=True)
    s = keep_ref[...] / jnp.maximum(jnp.sqrt(n2), 1e-8)
    xs = x_ref[...] * s
    part = lax.dot_general(xs, w, (((1,), (1,)), ((), ())),
                           preferred_element_type=jnp.float32)

    @pl.when(i == 0)
    def _init():
        o_ref[...] = part

    @pl.when(i > 0)
    def _acc():
        o_ref[...] += part


def _corr_body(u_ref, xg_ref, acc_ref, o_ref):
    i = pl.program_id(0)
    u = u_ref[...]
    n2 = jnp.sum(u * u, axis=0, keepdims=True)
    s = 1.0 / jnp.maximum(jnp.sqrt(n2), 1e-8)
    xs = xg_ref[...] * s
    part = lax.dot_general(xs, u, (((1,), (1,)), ((), ())),
                           preferred_element_type=jnp.float32)

    @pl.when(i == 0)
    def _init():
        o_ref[...] = acc_ref[...] + part

    @pl.when(i > 0)
    def _acc():
        o_ref[...] += part


def kernel(x, weight, dictionary_vector_indices, updated_weights):
    idx = dictionary_vector_indices.astype(jnp.int32)

    keep, xg = _route(idx, x)

    LB = 512
    nL = L // LB
    keep3 = keep.reshape(nL, 1, LB)
    acc1 = pl.pallas_call(
        _main_body,
        grid=(nL,),
        in_specs=[
            pl.BlockSpec((D, LB), lambda i: (0, i)),
            pl.BlockSpec((B, LB), lambda i: (0, i)),
            pl.BlockSpec((None, 1, LB), lambda i: (i, 0, 0)),
        ],
        out_specs=pl.BlockSpec((B, D), lambda i: (0, 0)),
        out_shape=jax.ShapeDtypeStruct((B, D), jnp.float32),
    )(weight, x, keep3)

    NB = 512
    nN = ND // NB
    out = pl.pallas_call(
        _corr_body,
        grid=(nN,),
        in_specs=[
            pl.BlockSpec((D, NB), lambda i: (0, i)),
            pl.BlockSpec((B, NB), lambda i: (0, i)),
            pl.BlockSpec((B, D), lambda i: (0, 0)),
        ],
        out_specs=pl.BlockSpec((B, D), lambda i: (0, 0)),
        out_shape=jax.ShapeDtypeStruct((B, D), jnp.float32),
    )(updated_weights, xg, acc1)
    return out
